# baseline (speedup 1.0000x reference)
import functools

import jax
import jax.numpy as jnp
from jax import lax
from jax.experimental import pallas as pl
from jax.experimental.pallas import tpu as pltpu
from jax.experimental.pallas import tpu_sc as plsc

NUM_CONCEPTS = 128
H1 = 512
H2 = 256
BATCH = 16384

NC = 2
NS = 16
NW = NC * NS
B_PER_W = BATCH // NW
CHUNK = 128
NCHUNK = B_PER_W // CHUNK
VPC = CHUNK * NUM_CONCEPTS // 16


def _sc_gather_combine(uid2, qid2, user_table, qdiff_table, qdisc_table,
                       Q_table):
  mesh = plsc.VectorSubcoreMesh(core_axis_name="c", subcore_axis_name="s")

  @functools.partial(
      pl.kernel,
      mesh=mesh,
      out_type=(
          jax.ShapeDtypeStruct((BATCH, NUM_CONCEPTS), jnp.float32),
          jax.ShapeDtypeStruct((BATCH,), jnp.float32),
      ),
      scratch_types=(
          pltpu.VMEM((NCHUNK, CHUNK), jnp.int32),
          pltpu.VMEM((NCHUNK, CHUNK), jnp.int32),
          pltpu.VMEM((CHUNK, NUM_CONCEPTS), jnp.float32),
          pltpu.VMEM((CHUNK, NUM_CONCEPTS), jnp.float32),
          pltpu.VMEM((CHUNK, NUM_CONCEPTS), jnp.float32),
          pltpu.VMEM((CHUNK,), jnp.float32),
          pltpu.SemaphoreType.DMA,
      ),
  )
  def k(uid_hbm, qid_hbm, ut_hbm, qd_hbm, qs_hbm, qm_hbm,
        x_out, s_out,
        uid_v, qid_v, ubuf, dbuf, qbuf, sbuf, sem):
    wid = lax.axis_index("s") * NC + lax.axis_index("c")
    pltpu.sync_copy(uid_hbm.at[pl.ds(wid * NCHUNK, NCHUNK)], uid_v)
    pltpu.sync_copy(qid_hbm.at[pl.ds(wid * NCHUNK, NCHUNK)], qid_v)
    for j in range(NCHUNK):
      base = wid * B_PER_W + j * CHUNK
      c1 = pltpu.async_copy(ut_hbm.at[uid_v.at[j]], ubuf, sem)
      c2 = pltpu.async_copy(qd_hbm.at[qid_v.at[j]], dbuf, sem)
      c3 = pltpu.async_copy(qm_hbm.at[qid_v.at[j]], qbuf, sem)
      c4 = pltpu.async_copy(qs_hbm.at[qid_v.at[j]], sbuf, sem)
      c1.wait(); c2.wait(); c3.wait(); c4.wait()

      def body(kk, carry):
        r = lax.shift_right_logical(kk, 3)
        c = (kk & 7) * 16
        u = ubuf[r, pl.ds(c, 16)]
        d = dbuf[r, pl.ds(c, 16)]
        q = qbuf[r, pl.ds(c, 16)]
        su = 1.0 / (1.0 + jnp.exp(-u))
        sd = 1.0 / (1.0 + jnp.exp(-d))
        ubuf[r, pl.ds(c, 16)] = (su - sd) * q
        return carry

      lax.fori_loop(0, VPC, body, 0)
      pltpu.sync_copy(ubuf, x_out.at[pl.ds(base, CHUNK)])
      pltpu.sync_copy(sbuf, s_out.at[pl.ds(base, CHUNK)])

  return k(uid2, qid2, user_table, qdiff_table, qdisc_table.reshape(-1),
           Q_table)


BT = 2048


def _sig(x):
  return 0.5 * jnp.tanh(0.5 * x) + 0.5


def _tc_mlp_body(x_ref, s_ref, w1_ref, b1_ref, w2_ref, b2_ref,
                 w3_ref, b3_ref, out_ref):
  disc = _sig(s_ref[...]) * 10.0
  x = disc * x_ref[...]
  h = _sig(
      jnp.dot(x, w1_ref[...], preferred_element_type=jnp.float32) + b1_ref[...])
  h = _sig(
      jnp.dot(h, w2_ref[...], preferred_element_type=jnp.float32) + b2_ref[...])
  o = _sig(
      jnp.dot(h, w3_ref[...], preferred_element_type=jnp.float32) + b3_ref[...])
  out_ref[...] = o


def _tc_mlp(x, s, W1, b1, W2, b2, W3, b3):
  grid = (BATCH // BT,)
  return pl.pallas_call(
      _tc_mlp_body,
      grid=grid,
      in_specs=[
          pl.BlockSpec((BT, NUM_CONCEPTS), lambda i: (i, 0)),
          pl.BlockSpec((BT, 1), lambda i: (i, 0)),
          pl.BlockSpec((NUM_CONCEPTS, H1), lambda i: (0, 0)),
          pl.BlockSpec((1, H1), lambda i: (0, 0)),
          pl.BlockSpec((H1, H2), lambda i: (0, 0)),
          pl.BlockSpec((1, H2), lambda i: (0, 0)),
          pl.BlockSpec((H2, 1), lambda i: (0, 0)),
          pl.BlockSpec((1, 1), lambda i: (0, 0)),
      ],
      out_specs=pl.BlockSpec((BT, 1), lambda i: (i, 0)),
      out_shape=jax.ShapeDtypeStruct((BATCH, 1), jnp.float32),
  )(x, s, W1, b1, W2, b2, W3, b3)


def kernel(user_id, question_id, user_table, qdiff_table, qdisc_table, Q_table,
           W1, b1, W2, b2, W3, b3):
  uid2 = user_id.astype(jnp.int32).reshape(BATCH // CHUNK, CHUNK)
  qid2 = question_id.astype(jnp.int32).reshape(BATCH // CHUNK, CHUNK)
  x, s = _sc_gather_combine(uid2, qid2, user_table, qdiff_table, qdisc_table,
                            Q_table)
  out = _tc_mlp(x, s.reshape(BATCH, 1), W1, b1.reshape(1, H1),
                W2, b2.reshape(1, H2), W3, b3.reshape(1, 1))
  return out.reshape(BATCH)


# speedup vs baseline: 2.1477x; 2.1477x over previous
import functools

import jax
import jax.numpy as jnp
from jax import lax
from jax.experimental import pallas as pl
from jax.experimental.pallas import tpu as pltpu
from jax.experimental.pallas import tpu_sc as plsc

NUM_CONCEPTS = 128
H1 = 512
H2 = 256
BATCH = 16384

NC = 2
NS = 16
NW = NC * NS
B_PER_W = BATCH // NW
CHUNK = 128
NCHUNK = B_PER_W // CHUNK
VPC = CHUNK * NUM_CONCEPTS // 16


def _sc_gather_combine(uid2, qid2, user_table, qdiff_table, qdisc_table,
                       Q_table):
  mesh = plsc.VectorSubcoreMesh(core_axis_name="c", subcore_axis_name="s")

  @functools.partial(
      pl.kernel,
      mesh=mesh,
      out_type=(
          jax.ShapeDtypeStruct((BATCH, NUM_CONCEPTS), jnp.float32),
          jax.ShapeDtypeStruct((BATCH,), jnp.float32),
      ),
      scratch_types=(
          pltpu.VMEM((NCHUNK, CHUNK), jnp.int32),
          pltpu.VMEM((NCHUNK, CHUNK), jnp.int32),
          pltpu.VMEM((2, CHUNK, NUM_CONCEPTS), jnp.float32),
          pltpu.VMEM((2, CHUNK, NUM_CONCEPTS), jnp.float32),
          pltpu.VMEM((2, CHUNK, NUM_CONCEPTS), jnp.float32),
          pltpu.VMEM((2, CHUNK), jnp.float32),
          (pltpu.SemaphoreType.DMA, pltpu.SemaphoreType.DMA),
          pltpu.SemaphoreType.DMA,
      ),
  )
  def k(uid_hbm, qid_hbm, ut_hbm, qd_hbm, qs_hbm, qm_hbm,
        x_out, s_out,
        uid_v, qid_v, ubuf, dbuf, qbuf, sbuf, gsems, wsem):
    wid = lax.axis_index("s") * NC + lax.axis_index("c")
    pltpu.sync_copy(uid_hbm.at[pl.ds(wid * NCHUNK, NCHUNK)], uid_v)
    pltpu.sync_copy(qid_hbm.at[pl.ds(wid * NCHUNK, NCHUNK)], qid_v)

    def fire(j, p):
      return (
          pltpu.async_copy(ut_hbm.at[uid_v.at[j]], ubuf.at[p], gsems[p]),
          pltpu.async_copy(qd_hbm.at[qid_v.at[j]], dbuf.at[p], gsems[p]),
          pltpu.async_copy(qm_hbm.at[qid_v.at[j]], qbuf.at[p], gsems[p]),
          pltpu.async_copy(qs_hbm.at[qid_v.at[j]], sbuf.at[p], gsems[p]),
      )

    pending = {0: fire(0, 0)}
    writes = []
    for j in range(NCHUNK):
      p = j % 2
      if j + 1 < NCHUNK:
        if j >= 1:
          writes.pop(0).wait()
          writes.pop(0).wait()
        pending[j + 1] = fire(j + 1, 1 - p)
      for cp in pending.pop(j):
        cp.wait()

      def body(r, carry, p=p):
        for ci in range(NUM_CONCEPTS // 16):
          sl = pl.ds(ci * 16, 16)
          a = jnp.exp(-ubuf[p, r, sl])
          b = jnp.exp(-dbuf[p, r, sl])
          q = qbuf[p, r, sl]
          ubuf[p, r, sl] = (b - a) / ((1.0 + a) * (1.0 + b)) * q
        return carry

      lax.fori_loop(0, CHUNK, body, 0)
      base = wid * B_PER_W + j * CHUNK
      writes.append(pltpu.async_copy(ubuf.at[p], x_out.at[pl.ds(base, CHUNK)],
                                     wsem))
      writes.append(pltpu.async_copy(sbuf.at[p], s_out.at[pl.ds(base, CHUNK)],
                                     wsem))
    for w in writes:
      w.wait()

  return k(uid2, qid2, user_table, qdiff_table, qdisc_table.reshape(-1),
           Q_table)


BT = 2048


def _sig(x):
  return 0.5 * jnp.tanh(0.5 * x) + 0.5


def _tc_mlp_body(x_ref, s_ref, w1_ref, b1_ref, w2_ref, b2_ref,
                 w3_ref, b3_ref, out_ref):
  disc = _sig(s_ref[...]) * 10.0
  x = disc * x_ref[...]
  h = _sig(
      jnp.dot(x, w1_ref[...], preferred_element_type=jnp.float32) + b1_ref[...])
  h = _sig(
      jnp.dot(h, w2_ref[...], preferred_element_type=jnp.float32) + b2_ref[...])
  o = _sig(
      jnp.dot(h, w3_ref[...], preferred_element_type=jnp.float32) + b3_ref[...])
  out_ref[...] = o


def _tc_mlp(x, s, W1, b1, W2, b2, W3, b3):
  grid = (BATCH // BT,)
  return pl.pallas_call(
      _tc_mlp_body,
      grid=grid,
      in_specs=[
          pl.BlockSpec((BT, NUM_CONCEPTS), lambda i: (i, 0)),
          pl.BlockSpec((BT, 1), lambda i: (i, 0)),
          pl.BlockSpec((NUM_CONCEPTS, H1), lambda i: (0, 0)),
          pl.BlockSpec((1, H1), lambda i: (0, 0)),
          pl.BlockSpec((H1, H2), lambda i: (0, 0)),
          pl.BlockSpec((1, H2), lambda i: (0, 0)),
          pl.BlockSpec((H2, 1), lambda i: (0, 0)),
          pl.BlockSpec((1, 1), lambda i: (0, 0)),
      ],
      out_specs=pl.BlockSpec((BT, 1), lambda i: (i, 0)),
      out_shape=jax.ShapeDtypeStruct((BATCH, 1), jnp.float32),
  )(x, s, W1, b1, W2, b2, W3, b3)


def kernel(user_id, question_id, user_table, qdiff_table, qdisc_table, Q_table,
           W1, b1, W2, b2, W3, b3):
  uid2 = user_id.astype(jnp.int32).reshape(BATCH // CHUNK, CHUNK)
  qid2 = question_id.astype(jnp.int32).reshape(BATCH // CHUNK, CHUNK)
  x, s = _sc_gather_combine(uid2, qid2, user_table, qdiff_table, qdisc_table,
                            Q_table)
  out = _tc_mlp(x, s.reshape(BATCH, 1), W1, b1.reshape(1, H1),
                W2, b2.reshape(1, H2), W3, b3.reshape(1, 1))
  return out.reshape(BATCH)


# speedup vs baseline: 2.3269x; 1.0834x over previous
import functools

import jax
import jax.numpy as jnp
from jax import lax
from jax.experimental import pallas as pl
from jax.experimental.pallas import tpu as pltpu
from jax.experimental.pallas import tpu_sc as plsc

NUM_CONCEPTS = 128
H1 = 512
H2 = 256
BATCH = 16384

NC = 2
NS = 16
NW = NC * NS
B_PER_W = BATCH // NW
CHUNK = 128
NCHUNK = B_PER_W // CHUNK
VPC = CHUNK * NUM_CONCEPTS // 16


def _sc_gather_combine(uid2, qid2, user_table, qdiff_table, qdisc_table,
                       Q_table):
  mesh = plsc.VectorSubcoreMesh(core_axis_name="c", subcore_axis_name="s")

  @functools.partial(
      pl.kernel,
      mesh=mesh,
      out_type=(
          jax.ShapeDtypeStruct((BATCH, NUM_CONCEPTS), jnp.float32),
          jax.ShapeDtypeStruct((BATCH,), jnp.float32),
      ),
      scratch_types=(
          pltpu.VMEM((NCHUNK, CHUNK), jnp.int32),
          pltpu.VMEM((NCHUNK, CHUNK), jnp.int32),
          pltpu.VMEM((2, CHUNK, NUM_CONCEPTS), jnp.float32),
          pltpu.VMEM((2, CHUNK, NUM_CONCEPTS), jnp.float32),
          pltpu.VMEM((2, CHUNK, NUM_CONCEPTS), jnp.float32),
          pltpu.VMEM((2, CHUNK), jnp.float32),
          (pltpu.SemaphoreType.DMA, pltpu.SemaphoreType.DMA),
          pltpu.SemaphoreType.DMA,
      ),
  )
  def k(uid_hbm, qid_hbm, ut_hbm, qd_hbm, qs_hbm, qm_hbm,
        x_out, s_out,
        uid_v, qid_v, ubuf, dbuf, qbuf, sbuf, gsems, wsem):
    wid = lax.axis_index("s") * NC + lax.axis_index("c")
    pltpu.sync_copy(uid_hbm.at[pl.ds(wid * NCHUNK, NCHUNK)], uid_v)
    pltpu.sync_copy(qid_hbm.at[pl.ds(wid * NCHUNK, NCHUNK)], qid_v)

    def fire(j, p):
      return (
          pltpu.async_copy(ut_hbm.at[uid_v.at[j]], ubuf.at[p], gsems[p]),
          pltpu.async_copy(qd_hbm.at[qid_v.at[j]], dbuf.at[p], gsems[p]),
          pltpu.async_copy(qm_hbm.at[qid_v.at[j]], qbuf.at[p], gsems[p]),
          pltpu.async_copy(qs_hbm.at[qid_v.at[j]], sbuf.at[p], gsems[p]),
      )

    pending = {0: fire(0, 0)}
    writes = []
    for j in range(NCHUNK):
      p = j % 2
      if j + 1 < NCHUNK:
        if j >= 1:
          writes.pop(0).wait()
          writes.pop(0).wait()
        pending[j + 1] = fire(j + 1, 1 - p)
      for cp in pending.pop(j):
        cp.wait()

      def body(r, carry, p=p):
        for ci in range(NUM_CONCEPTS // 16):
          sl = pl.ds(ci * 16, 16)
          a = jnp.exp(-ubuf[p, r, sl])
          b = jnp.exp(-dbuf[p, r, sl])
          q = qbuf[p, r, sl]
          ubuf[p, r, sl] = (b - a) / ((1.0 + a) * (1.0 + b)) * q
        return carry

      lax.fori_loop(0, CHUNK, body, 0)
      base = wid * B_PER_W + j * CHUNK
      writes.append(pltpu.async_copy(ubuf.at[p], x_out.at[pl.ds(base, CHUNK)],
                                     wsem))
      writes.append(pltpu.async_copy(sbuf.at[p], s_out.at[pl.ds(base, CHUNK)],
                                     wsem))
    for w in writes:
      w.wait()

  return k(uid2, qid2, user_table, qdiff_table, qdisc_table.reshape(-1),
           Q_table)


BT = 2048


def _sig(x):
  return 0.5 * jnp.tanh(0.5 * x) + 0.5


def _tc_mlp_body(x_ref, s_ref, w1_ref, b1_ref, w2_ref, b2_ref,
                 w3_ref, b3_ref, out_ref):
  disc = _sig(s_ref[...].reshape(BT, 1)) * 10.0
  x = disc * x_ref[...]
  h = _sig(
      jnp.dot(x, w1_ref[...], preferred_element_type=jnp.float32) + b1_ref[...])
  h = _sig(
      jnp.dot(h, w2_ref[...], preferred_element_type=jnp.float32) + b2_ref[...])
  o = _sig(
      jnp.dot(h, w3_ref[...], preferred_element_type=jnp.float32) + b3_ref[...])
  out_ref[...] = o.reshape(BT)


def _tc_mlp(x, s, W1, b1, W2, b2, W3, b3):
  grid = (BATCH // BT,)
  return pl.pallas_call(
      _tc_mlp_body,
      grid=grid,
      in_specs=[
          pl.BlockSpec((BT, NUM_CONCEPTS), lambda i: (i, 0)),
          pl.BlockSpec((1, 1, BT), lambda i: (i, 0, 0)),
          pl.BlockSpec((NUM_CONCEPTS, H1), lambda i: (0, 0)),
          pl.BlockSpec((1, H1), lambda i: (0, 0)),
          pl.BlockSpec((H1, H2), lambda i: (0, 0)),
          pl.BlockSpec((1, H2), lambda i: (0, 0)),
          pl.BlockSpec((H2, 1), lambda i: (0, 0)),
          pl.BlockSpec((1, 1), lambda i: (0, 0)),
      ],
      out_specs=pl.BlockSpec((BT,), lambda i: (i,)),
      out_shape=jax.ShapeDtypeStruct((BATCH,), jnp.float32),
  )(x, s, W1, b1, W2, b2, W3, b3)


def kernel(user_id, question_id, user_table, qdiff_table, qdisc_table, Q_table,
           W1, b1, W2, b2, W3, b3):
  uid2 = user_id.astype(jnp.int32).reshape(BATCH // CHUNK, CHUNK)
  qid2 = question_id.astype(jnp.int32).reshape(BATCH // CHUNK, CHUNK)
  x, s = _sc_gather_combine(uid2, qid2, user_table, qdiff_table, qdisc_table,
                            Q_table)
  return _tc_mlp(x, s.reshape(BATCH // BT, 1, BT), W1, b1.reshape(1, H1),
                 W2, b2.reshape(1, H2), W3, b3.reshape(1, 1))
